# Initial kernel scaffold; baseline (speedup 1.0000x reference)
import jax
import jax.numpy as jnp
from jax.experimental import pallas as pl


def kernel(h, e, graph, edge_index, sparse, WU, bU, WV, bV, WA, bA, WB, bB, WC, bC, gamma_h, beta_h, gamma_e, beta_e):
    raise NotImplementedError("write your pallas kernel here")


# speedup vs baseline: 1.2859x; 1.2859x over previous
import functools

import jax
import jax.numpy as jnp
from jax import lax
from jax.experimental import pallas as pl
from jax.experimental.pallas import tpu as pltpu
from jax.experimental.pallas import tpu_sc as plsc

V = 10000
E = 320000
H = 128
HV = 2 * H

NC = 2
NS = 16
NW = NC * NS
EPW = E // NW
K = 40
NCHUNK = EPW // K

_f32 = jnp.float32


_NODE_BLK = 2000
_EDGE_BLK = 2000


def _node_mm_body(h_ref, wu_ref, bu_ref, wb_ref, bb_ref, wav_ref, bav_ref,
                  uh_ref, bh_ref, av_ref):
    hb = h_ref[...]
    uh_ref[...] = jnp.dot(hb, wu_ref[...], preferred_element_type=_f32) + bu_ref[...]
    bh_ref[...] = jnp.dot(hb, wb_ref[...], preferred_element_type=_f32) + bb_ref[...]
    av_ref[...] = jnp.dot(hb, wav_ref[...], preferred_element_type=_f32) + bav_ref[...]


def _node_mm(h, WU, bU, WB, bB, WAV, bAV):
    grid = (V // _NODE_BLK,)
    return pl.pallas_call(
        _node_mm_body,
        grid=grid,
        in_specs=[
            pl.BlockSpec((_NODE_BLK, H), lambda i: (i, i * 0)),
            pl.BlockSpec((H, H), lambda i: (i * 0, i * 0)),
            pl.BlockSpec((1, H), lambda i: (i * 0, i * 0)),
            pl.BlockSpec((H, H), lambda i: (i * 0, i * 0)),
            pl.BlockSpec((1, H), lambda i: (i * 0, i * 0)),
            pl.BlockSpec((H, HV), lambda i: (i * 0, i * 0)),
            pl.BlockSpec((1, HV), lambda i: (i * 0, i * 0)),
        ],
        out_specs=[
            pl.BlockSpec((_NODE_BLK, H), lambda i: (i, i * 0)),
            pl.BlockSpec((_NODE_BLK, H), lambda i: (i, i * 0)),
            pl.BlockSpec((_NODE_BLK, HV), lambda i: (i, i * 0)),
        ],
        out_shape=[
            jax.ShapeDtypeStruct((V, H), _f32),
            jax.ShapeDtypeStruct((V, H), _f32),
            jax.ShapeDtypeStruct((V, HV), _f32),
        ],
    )(h, WU, bU, WB, bB, WAV, bAV)


def _ce_body(e_ref, wc_ref, bc_ref, ce_ref):
    ce_ref[...] = jnp.dot(e_ref[...], wc_ref[...], preferred_element_type=_f32) + bc_ref[...]


def _ce_mm(e, WC, bC):
    grid = (E // _EDGE_BLK,)
    return pl.pallas_call(
        _ce_body,
        grid=grid,
        in_specs=[
            pl.BlockSpec((_EDGE_BLK, H), lambda i: (i, i * 0)),
            pl.BlockSpec((H, H), lambda i: (i * 0, i * 0)),
            pl.BlockSpec((1, H), lambda i: (i * 0, i * 0)),
        ],
        out_specs=pl.BlockSpec((_EDGE_BLK, H), lambda i: (i, i * 0)),
        out_shape=jax.ShapeDtypeStruct((E, H), _f32),
    )(e, WC, bC)


def _ln_relu_res(x, xnew, g, b):
    mu = jnp.mean(xnew, axis=-1, keepdims=True)
    var = jnp.mean((xnew - mu) ** 2, axis=-1, keepdims=True)
    ln = (xnew - mu) * lax.rsqrt(var + 1e-5) * g + b
    return x + jnp.maximum(ln, 0.0)


def _h_final_body(h_ref, uh_ref, a0_ref, a1_ref, g_ref, b_ref, out_ref):
    hn = uh_ref[...] + a0_ref[0] + a1_ref[0]
    out_ref[...] = _ln_relu_res(h_ref[...], hn, g_ref[...], b_ref[...])


def _h_final(h, uh, agg, gamma, beta):
    grid = (V // _NODE_BLK,)
    return pl.pallas_call(
        _h_final_body,
        grid=grid,
        in_specs=[
            pl.BlockSpec((_NODE_BLK, H), lambda i: (i, i * 0)),
            pl.BlockSpec((_NODE_BLK, H), lambda i: (i, i * 0)),
            pl.BlockSpec((1, _NODE_BLK, H), lambda i: (i * 0, i, i * 0)),
            pl.BlockSpec((1, _NODE_BLK, H), lambda i: (i * 0 + 1, i, i * 0)),
            pl.BlockSpec((1, H), lambda i: (i * 0, i * 0)),
            pl.BlockSpec((1, H), lambda i: (i * 0, i * 0)),
        ],
        out_specs=pl.BlockSpec((_NODE_BLK, H), lambda i: (i, i * 0)),
        out_shape=jax.ShapeDtypeStruct((V, H), _f32),
    )(h, uh, agg, agg, gamma, beta)


def _e_final_body(e_ref, en_ref, g_ref, b_ref, out_ref):
    out_ref[...] = _ln_relu_res(e_ref[...], en_ref[...], g_ref[...], b_ref[...])


def _e_final(e, enew, gamma, beta):
    grid = (E // _EDGE_BLK,)
    return pl.pallas_call(
        _e_final_body,
        grid=grid,
        in_specs=[
            pl.BlockSpec((_EDGE_BLK, H), lambda i: (i, i * 0)),
            pl.BlockSpec((_EDGE_BLK, H), lambda i: (i, i * 0)),
            pl.BlockSpec((1, H), lambda i: (i * 0, i * 0)),
            pl.BlockSpec((1, H), lambda i: (i * 0, i * 0)),
        ],
        out_specs=pl.BlockSpec((_EDGE_BLK, H), lambda i: (i, i * 0)),
        out_shape=jax.ShapeDtypeStruct((E, H), _f32),
    )(e, enew, gamma, beta)


_sc_mesh = plsc.VectorSubcoreMesh(core_axis_name="c", subcore_axis_name="s")


@functools.partial(
    pl.kernel,
    out_type=[
        jax.ShapeDtypeStruct((E, H), _f32),
        jax.ShapeDtypeStruct((NC, V, H), _f32),
    ],
    mesh=_sc_mesh,
    scratch_types=[
        pltpu.VMEM((K,), jnp.int32),
        pltpu.VMEM((K,), jnp.int32),
        pltpu.VMEM((K, HV), _f32),
        pltpu.VMEM((K, H), _f32),
        pltpu.VMEM((K, H), _f32),
        pltpu.VMEM((K, H), _f32),
        pltpu.VMEM_SHARED((V, H), _f32),
        pltpu.SemaphoreType.DMA,
        pltpu.SemaphoreType.DMA,
    ],
)
def _sc_edge_kernel(av_hbm, bh_hbm, ce_hbm, src_hbm, dst_hbm, zeros_hbm,
                    enew_hbm, agg_hbm,
                    srcv, dstv, avv, bhv, cev, contribv, aggs,
                    sem1, sem2):
    cid = lax.axis_index("c")
    sid = lax.axis_index("s")
    wid = sid * jnp.int32(NC) + cid

    @pl.when(sid == 0)
    def _zero():
        pltpu.sync_copy(zeros_hbm, aggs)

    plsc.subcore_barrier()

    base0 = wid * jnp.int32(EPW)

    def chunk_body(i, carry):
        base = base0 + i * jnp.int32(K)
        pltpu.sync_copy(src_hbm.at[pl.ds(base, K)], srcv)
        pltpu.sync_copy(dst_hbm.at[pl.ds(base, K)], dstv)
        cp_av = pltpu.async_copy(av_hbm.at[dstv], avv, sem1)
        cp_bh = pltpu.async_copy(bh_hbm.at[srcv], bhv, sem2)
        pltpu.sync_copy(ce_hbm.at[pl.ds(base, K)], cev)
        cp_av.wait()
        cp_bh.wait()

        def edge_body(j, c2):
            for q in range(H // 16):
                sl = pl.ds(q * 16, 16)
                en = avv[j, sl] + bhv[j, sl] + cev[j, sl]
                cev[j, sl] = en
                gate = 1.0 / (1.0 + jnp.exp(-en))
                contribv[j, sl] = gate * avv[j, pl.ds(H + q * 16, 16)]
            return c2

        lax.fori_loop(jnp.int32(0), jnp.int32(K), edge_body, jnp.int32(0))
        pltpu.sync_copy(cev, enew_hbm.at[pl.ds(base, K)])
        pltpu.sync_copy(contribv, aggs.at[srcv], add=True)
        return carry

    lax.fori_loop(jnp.int32(0), jnp.int32(NCHUNK), chunk_body, jnp.int32(0))

    plsc.subcore_barrier()

    @pl.when(sid == 0)
    def _dump():
        pltpu.sync_copy(aggs, agg_hbm.at[cid])


def kernel(h, e, graph, edge_index, sparse,
           WU, bU, WV, bV, WA, bA, WB, bB, WC, bC,
           gamma_h, beta_h, gamma_e, beta_e):
    src = edge_index[0].astype(jnp.int32)
    dst = edge_index[1].astype(jnp.int32)

    WAV = jnp.concatenate([WA, WV], axis=1)
    bAV = jnp.concatenate([bA, bV]).reshape(1, HV)
    bU2 = bU.reshape(1, H)
    bB2 = bB.reshape(1, H)
    bC2 = bC.reshape(1, H)

    uh, bh, av = _node_mm(h, WU, bU2, WB, bB2, WAV, bAV)
    ce = _ce_mm(e, WC, bC2)

    zeros = jnp.zeros((V, H), _f32)
    enew, agg = _sc_edge_kernel(av, bh, ce, src, dst, zeros)

    h_out = _h_final(h, uh, agg, gamma_h.reshape(1, H), beta_h.reshape(1, H))
    e_out = _e_final(e, enew, gamma_e.reshape(1, H), beta_e.reshape(1, H))
    return (h_out, e_out)


# speedup vs baseline: 4.2952x; 3.3401x over previous
import functools

import jax
import jax.numpy as jnp
from jax import lax
from jax.experimental import pallas as pl
from jax.experimental.pallas import tpu as pltpu
from jax.experimental.pallas import tpu_sc as plsc

V = 10000
E = 320000
H = 128
H4 = 4 * H

NC = 2
NS = 16
NW = NC * NS
EPW = E // NW
K = 40
NCHUNK = EPW // K
G = 10
NSUP = NCHUNK // G
ROWS = E // K

_f32 = jnp.float32


_NODE_BLK = 2000
_EDGE_BLK = 2000


def _node_mm_body(h_ref, w4_ref, b4_ref, uh_ref, ah_ref, bh_ref, vh_ref):
    hb = h_ref[...]
    out = jnp.dot(hb, w4_ref[...], preferred_element_type=_f32) + b4_ref[...]
    uh_ref[...] = out[:, 0 * H:1 * H]
    ah_ref[...] = out[:, 1 * H:2 * H]
    bh_ref[...] = out[:, 2 * H:3 * H]
    vh_ref[...] = out[:, 3 * H:4 * H]


def _node_mm(h, W4, b4):
    grid = (V // _NODE_BLK,)
    spec = pl.BlockSpec((_NODE_BLK, H), lambda i: (i, i * 0))
    return pl.pallas_call(
        _node_mm_body,
        grid=grid,
        in_specs=[
            pl.BlockSpec((_NODE_BLK, H), lambda i: (i, i * 0)),
            pl.BlockSpec((H, H4), lambda i: (i * 0, i * 0)),
            pl.BlockSpec((1, H4), lambda i: (i * 0, i * 0)),
        ],
        out_specs=[spec, spec, spec, spec],
        out_shape=[jax.ShapeDtypeStruct((V, H), _f32)] * 4,
    )(h, W4, b4)


def _ce_body(e_ref, wc_ref, bc_ref, ce_ref):
    ce_ref[...] = jnp.dot(e_ref[...], wc_ref[...], preferred_element_type=_f32) + bc_ref[...]


def _ce_mm(e, WC, bC):
    grid = (E // _EDGE_BLK,)
    return pl.pallas_call(
        _ce_body,
        grid=grid,
        in_specs=[
            pl.BlockSpec((_EDGE_BLK, H), lambda i: (i, i * 0)),
            pl.BlockSpec((H, H), lambda i: (i * 0, i * 0)),
            pl.BlockSpec((1, H), lambda i: (i * 0, i * 0)),
        ],
        out_specs=pl.BlockSpec((_EDGE_BLK, H), lambda i: (i, i * 0)),
        out_shape=jax.ShapeDtypeStruct((E, H), _f32),
    )(e, WC, bC)


def _ln_relu_res(x, xnew, g, b):
    mu = jnp.mean(xnew, axis=-1, keepdims=True)
    var = jnp.mean((xnew - mu) ** 2, axis=-1, keepdims=True)
    ln = (xnew - mu) * lax.rsqrt(var + 1e-5) * g + b
    return x + jnp.maximum(ln, 0.0)


def _h_final_body(h_ref, uh_ref, a0_ref, a1_ref, g_ref, b_ref, out_ref):
    hn = uh_ref[...] + a0_ref[0] + a1_ref[0]
    out_ref[...] = _ln_relu_res(h_ref[...], hn, g_ref[...], b_ref[...])


def _h_final(h, uh, agg, gamma, beta):
    grid = (V // _NODE_BLK,)
    return pl.pallas_call(
        _h_final_body,
        grid=grid,
        in_specs=[
            pl.BlockSpec((_NODE_BLK, H), lambda i: (i, i * 0)),
            pl.BlockSpec((_NODE_BLK, H), lambda i: (i, i * 0)),
            pl.BlockSpec((1, _NODE_BLK, H), lambda i: (i * 0, i, i * 0)),
            pl.BlockSpec((1, _NODE_BLK, H), lambda i: (i * 0 + 1, i, i * 0)),
            pl.BlockSpec((1, H), lambda i: (i * 0, i * 0)),
            pl.BlockSpec((1, H), lambda i: (i * 0, i * 0)),
        ],
        out_specs=pl.BlockSpec((_NODE_BLK, H), lambda i: (i, i * 0)),
        out_shape=jax.ShapeDtypeStruct((V, H), _f32),
    )(h, uh, agg, agg, gamma, beta)


def _e_final_body(e_ref, en_ref, g_ref, b_ref, out_ref):
    out_ref[...] = _ln_relu_res(e_ref[...], en_ref[...], g_ref[...], b_ref[...])


def _e_final(e, enew, gamma, beta):
    grid = (E // _EDGE_BLK,)
    return pl.pallas_call(
        _e_final_body,
        grid=grid,
        in_specs=[
            pl.BlockSpec((_EDGE_BLK, H), lambda i: (i, i * 0)),
            pl.BlockSpec((_EDGE_BLK, H), lambda i: (i, i * 0)),
            pl.BlockSpec((1, H), lambda i: (i * 0, i * 0)),
            pl.BlockSpec((1, H), lambda i: (i * 0, i * 0)),
        ],
        out_specs=pl.BlockSpec((_EDGE_BLK, H), lambda i: (i, i * 0)),
        out_shape=jax.ShapeDtypeStruct((E, H), _f32),
    )(e, enew, gamma, beta)


_sc_mesh = plsc.VectorSubcoreMesh(core_axis_name="c", subcore_axis_name="s")


@functools.partial(
    pl.kernel,
    out_type=[
        jax.ShapeDtypeStruct((E, H), _f32),
        jax.ShapeDtypeStruct((NC, V, H), _f32),
    ],
    mesh=_sc_mesh,
    scratch_types=[
        pltpu.VMEM((G, K), jnp.int32),
        pltpu.VMEM((G, K), jnp.int32),
        pltpu.VMEM((2, K, H), _f32),
        pltpu.VMEM((2, K, H), _f32),
        pltpu.VMEM((2, K, H), _f32),
        pltpu.VMEM((2, K, H), _f32),
        pltpu.VMEM_SHARED((V, H), _f32),
        pltpu.SemaphoreType.DMA,
        pltpu.SemaphoreType.DMA,
        pltpu.SemaphoreType.DMA,
        pltpu.SemaphoreType.DMA,
        pltpu.SemaphoreType.DMA,
    ],
)
def _sc_edge_kernel(ah_hbm, vh_hbm, bh_hbm, ce_hbm, src_hbm, dst_hbm, zeros_hbm,
                    enew_hbm, agg_hbm,
                    srcv, dstv, ahv, vhv, bhv, cev, aggs,
                    si0, si1, so0, so1, sem_idx):
    cid = lax.axis_index("c")
    sid = lax.axis_index("s")
    wid = sid * jnp.int32(NC) + cid
    sem_in = (si0, si1)
    sem_out = (so0, so1)

    @pl.when(sid == 0)
    def _zero():
        pltpu.sync_copy(zeros_hbm, aggs)

    plsc.subcore_barrier()

    base0 = wid * jnp.int32(EPW)

    z32 = jnp.int32(0)

    def issue_in(base, c, b):
        ci = jnp.int32(c)
        bi = jnp.int32(b)
        cps = (
            pltpu.make_async_copy(ah_hbm.at[dstv.at[ci]], ahv.at[bi], sem_in[b]),
            pltpu.make_async_copy(vh_hbm.at[dstv.at[ci]], vhv.at[bi], sem_in[b]),
            pltpu.make_async_copy(bh_hbm.at[srcv.at[ci]], bhv.at[bi], sem_in[b]),
            pltpu.make_async_copy(ce_hbm.at[pl.ds(base, K)], cev.at[bi], sem_in[b]),
        )
        for cp in cps:
            cp.start()
        return cps

    def issue_out(base, c, b):
        ci = jnp.int32(c)
        bi = jnp.int32(b)
        cps = (
            pltpu.make_async_copy(cev.at[bi], enew_hbm.at[pl.ds(base, K)], sem_out[b]),
        )
        cps[0].start()
        pltpu.sync_copy(vhv.at[bi], aggs.at[srcv.at[ci]], add=True)
        return cps

    def compute(b):
        bi = jnp.int32(b)

        def _cbody(j, carry):
            for q8 in range(H // 16):
                sl = pl.ds(q8 * 16, 16)
                en = ahv[bi, j, sl] + bhv[bi, j, sl] + cev[bi, j, sl]
                cev[bi, j, sl] = en
                gate = 1.0 / (1.0 + jnp.exp(-en))
                vhv[bi, j, sl] = gate * vhv[bi, j, sl]
            return carry

        lax.fori_loop(jnp.int32(0), jnp.int32(K), _cbody, jnp.int32(0))

    def super_body(s, carry):
        base_s = base0 + s * jnp.int32(G * K)
        idx_cps = []
        for c in range(G):
            ci = jnp.int32(c)
            off = base_s + c * K
            idx_cps.append(pltpu.make_async_copy(
                src_hbm.at[pl.ds(off, K)], srcv.at[ci], sem_idx))
            idx_cps.append(pltpu.make_async_copy(
                dst_hbm.at[pl.ds(off, K)], dstv.at[ci], sem_idx))
        for cp in idx_cps:
            cp.start()
        for cp in idx_cps:
            cp.wait()

        d_in = [None] * G
        d_out = [None] * G
        d_in[0] = issue_in(base_s, 0, 0)
        for c in range(G):
            p = c % 2
            base = base_s + c * K
            if c < G - 1:
                if c >= 1:
                    for w in d_out[c - 1]:
                        w.wait()
                d_in[c + 1] = issue_in(base + K, c + 1, 1 - p)
            for w in d_in[c]:
                w.wait()
            compute(p)
            d_out[c] = issue_out(base, c, p)
        for w in d_out[G - 2]:
            w.wait()
        for w in d_out[G - 1]:
            w.wait()
        return carry

    lax.fori_loop(jnp.int32(0), jnp.int32(NSUP), super_body, jnp.int32(0))

    plsc.subcore_barrier()

    @pl.when(sid == 0)
    def _dump():
        pltpu.sync_copy(aggs, agg_hbm.at[cid])


def kernel(h, e, graph, edge_index, sparse,
           WU, bU, WV, bV, WA, bA, WB, bB, WC, bC,
           gamma_h, beta_h, gamma_e, beta_e):
    src = edge_index[0].astype(jnp.int32)
    dst = edge_index[1].astype(jnp.int32)

    W4 = jnp.concatenate([WU, WA, WB, WV], axis=1)
    b4 = jnp.concatenate([bU, bA, bB, bV]).reshape(1, H4)
    bC2 = bC.reshape(1, H)

    uh, ah, bh, vh = _node_mm(h, W4, b4)
    ce = _ce_mm(e, WC, bC2)

    zeros = jnp.zeros((V, H), _f32)
    enew, agg = _sc_edge_kernel(ah, vh, bh, ce, src, dst, zeros)

    h_out = _h_final(h, uh, agg, gamma_h.reshape(1, H), beta_h.reshape(1, H))
    e_out = _e_final(e, enew, gamma_e.reshape(1, H), beta_e.reshape(1, H))
    return (h_out, e_out)


# speedup vs baseline: 4.4643x; 1.0394x over previous
import functools

import jax
import jax.numpy as jnp
from jax import lax
from jax.experimental import pallas as pl
from jax.experimental.pallas import tpu as pltpu
from jax.experimental.pallas import tpu_sc as plsc

V = 10000
E = 320000
H = 128
H4 = 4 * H

NC = 2
NS = 16
NW = NC * NS
EPW = E // NW
K = 40
NCHUNK = EPW // K
G = 25
NSUP = NCHUNK // G
ROWS = E // K

_f32 = jnp.float32


_NODE_BLK = 2000
_EDGE_BLK = 2000


def _node_mm_body(h_ref, w4_ref, b4_ref, uh_ref, ah_ref, bh_ref, vh_ref):
    hb = h_ref[...]
    out = jnp.dot(hb, w4_ref[...], preferred_element_type=_f32) + b4_ref[...]
    uh_ref[...] = out[:, 0 * H:1 * H]
    ah_ref[...] = out[:, 1 * H:2 * H]
    bh_ref[...] = out[:, 2 * H:3 * H]
    vh_ref[...] = out[:, 3 * H:4 * H]


def _node_mm(h, W4, b4):
    grid = (V // _NODE_BLK,)
    spec = pl.BlockSpec((_NODE_BLK, H), lambda i: (i, i * 0))
    return pl.pallas_call(
        _node_mm_body,
        grid=grid,
        in_specs=[
            pl.BlockSpec((_NODE_BLK, H), lambda i: (i, i * 0)),
            pl.BlockSpec((H, H4), lambda i: (i * 0, i * 0)),
            pl.BlockSpec((1, H4), lambda i: (i * 0, i * 0)),
        ],
        out_specs=[spec, spec, spec, spec],
        out_shape=[jax.ShapeDtypeStruct((V, H), _f32)] * 4,
    )(h, W4, b4)


def _ce_body(e_ref, wc_ref, bc_ref, ce_ref):
    ce_ref[...] = jnp.dot(e_ref[...], wc_ref[...], preferred_element_type=_f32) + bc_ref[...]


def _ce_mm(e, WC, bC):
    grid = (E // _EDGE_BLK,)
    return pl.pallas_call(
        _ce_body,
        grid=grid,
        in_specs=[
            pl.BlockSpec((_EDGE_BLK, H), lambda i: (i, i * 0)),
            pl.BlockSpec((H, H), lambda i: (i * 0, i * 0)),
            pl.BlockSpec((1, H), lambda i: (i * 0, i * 0)),
        ],
        out_specs=pl.BlockSpec((_EDGE_BLK, H), lambda i: (i, i * 0)),
        out_shape=jax.ShapeDtypeStruct((E, H), _f32),
    )(e, WC, bC)


def _ln_relu_res(x, xnew, g, b):
    mu = jnp.mean(xnew, axis=-1, keepdims=True)
    var = jnp.mean((xnew - mu) ** 2, axis=-1, keepdims=True)
    ln = (xnew - mu) * lax.rsqrt(var + 1e-5) * g + b
    return x + jnp.maximum(ln, 0.0)


def _h_final_body(h_ref, uh_ref, a0_ref, a1_ref, g_ref, b_ref, out_ref):
    hn = uh_ref[...] + a0_ref[0] + a1_ref[0]
    out_ref[...] = _ln_relu_res(h_ref[...], hn, g_ref[...], b_ref[...])


def _h_final(h, uh, agg, gamma, beta):
    grid = (V // _NODE_BLK,)
    return pl.pallas_call(
        _h_final_body,
        grid=grid,
        in_specs=[
            pl.BlockSpec((_NODE_BLK, H), lambda i: (i, i * 0)),
            pl.BlockSpec((_NODE_BLK, H), lambda i: (i, i * 0)),
            pl.BlockSpec((1, _NODE_BLK, H), lambda i: (i * 0, i, i * 0)),
            pl.BlockSpec((1, _NODE_BLK, H), lambda i: (i * 0 + 1, i, i * 0)),
            pl.BlockSpec((1, H), lambda i: (i * 0, i * 0)),
            pl.BlockSpec((1, H), lambda i: (i * 0, i * 0)),
        ],
        out_specs=pl.BlockSpec((_NODE_BLK, H), lambda i: (i, i * 0)),
        out_shape=jax.ShapeDtypeStruct((V, H), _f32),
    )(h, uh, agg, agg, gamma, beta)


def _e_final_body(e_ref, en_ref, g_ref, b_ref, out_ref):
    out_ref[...] = _ln_relu_res(e_ref[...], en_ref[...], g_ref[...], b_ref[...])


def _e_final(e, enew, gamma, beta):
    grid = (E // _EDGE_BLK,)
    return pl.pallas_call(
        _e_final_body,
        grid=grid,
        in_specs=[
            pl.BlockSpec((_EDGE_BLK, H), lambda i: (i, i * 0)),
            pl.BlockSpec((_EDGE_BLK, H), lambda i: (i, i * 0)),
            pl.BlockSpec((1, H), lambda i: (i * 0, i * 0)),
            pl.BlockSpec((1, H), lambda i: (i * 0, i * 0)),
        ],
        out_specs=pl.BlockSpec((_EDGE_BLK, H), lambda i: (i, i * 0)),
        out_shape=jax.ShapeDtypeStruct((E, H), _f32),
    )(e, enew, gamma, beta)


_sc_mesh = plsc.VectorSubcoreMesh(core_axis_name="c", subcore_axis_name="s")


@functools.partial(
    pl.kernel,
    out_type=[
        jax.ShapeDtypeStruct((E, H), _f32),
        jax.ShapeDtypeStruct((NC, V, H), _f32),
    ],
    mesh=_sc_mesh,
    scratch_types=[
        pltpu.VMEM((G, K), jnp.int32),
        pltpu.VMEM((G, K), jnp.int32),
        pltpu.VMEM((2, K, H), _f32),
        pltpu.VMEM((2, K, H), _f32),
        pltpu.VMEM((2, K, H), _f32),
        pltpu.VMEM((2, K, H), _f32),
        pltpu.VMEM_SHARED((V, H), _f32),
        pltpu.SemaphoreType.DMA,
        pltpu.SemaphoreType.DMA,
        pltpu.SemaphoreType.DMA,
        pltpu.SemaphoreType.DMA,
        pltpu.SemaphoreType.DMA,
    ],
)
def _sc_edge_kernel(ah_hbm, vh_hbm, bh_hbm, ce_hbm, src_hbm, dst_hbm, zeros_hbm,
                    enew_hbm, agg_hbm,
                    srcv, dstv, ahv, vhv, bhv, cev, aggs,
                    si0, si1, so0, so1, sem_idx):
    cid = lax.axis_index("c")
    sid = lax.axis_index("s")
    wid = sid * jnp.int32(NC) + cid
    sem_in = (si0, si1)
    sem_out = (so0, so1)

    @pl.when(sid == 0)
    def _zero():
        pltpu.sync_copy(zeros_hbm, aggs)

    plsc.subcore_barrier()

    base0 = wid * jnp.int32(EPW)

    z32 = jnp.int32(0)

    def issue_in(base, c, b):
        ci = jnp.int32(c)
        bi = jnp.int32(b)
        cps = (
            pltpu.make_async_copy(ah_hbm.at[dstv.at[ci]], ahv.at[bi], sem_in[b]),
            pltpu.make_async_copy(vh_hbm.at[dstv.at[ci]], vhv.at[bi], sem_in[b]),
            pltpu.make_async_copy(bh_hbm.at[srcv.at[ci]], bhv.at[bi], sem_in[b]),
            pltpu.make_async_copy(ce_hbm.at[pl.ds(base, K)], cev.at[bi], sem_in[b]),
        )
        for cp in cps:
            cp.start()
        return cps

    def issue_out(base, c, b):
        ci = jnp.int32(c)
        bi = jnp.int32(b)
        cps = (
            pltpu.make_async_copy(cev.at[bi], enew_hbm.at[pl.ds(base, K)], sem_out[b]),
        )
        cps[0].start()
        pltpu.sync_copy(vhv.at[bi], aggs.at[srcv.at[ci]], add=True)
        return cps

    def compute(b):
        bi = jnp.int32(b)

        def _cbody(j, carry):
            for q8 in range(H // 16):
                sl = pl.ds(q8 * 16, 16)
                en = ahv[bi, j, sl] + bhv[bi, j, sl] + cev[bi, j, sl]
                cev[bi, j, sl] = en
                gate = 1.0 / (1.0 + jnp.exp(-en))
                vhv[bi, j, sl] = gate * vhv[bi, j, sl]
            return carry

        lax.fori_loop(jnp.int32(0), jnp.int32(K), _cbody, jnp.int32(0))

    def super_body(s, carry):
        base_s = base0 + s * jnp.int32(G * K)
        idx_cps = []
        for c in range(G):
            ci = jnp.int32(c)
            off = base_s + c * K
            idx_cps.append(pltpu.make_async_copy(
                src_hbm.at[pl.ds(off, K)], srcv.at[ci], sem_idx))
            idx_cps.append(pltpu.make_async_copy(
                dst_hbm.at[pl.ds(off, K)], dstv.at[ci], sem_idx))
        for cp in idx_cps:
            cp.start()
        for cp in idx_cps:
            cp.wait()

        d_in = [None] * G
        d_out = [None] * G
        d_in[0] = issue_in(base_s, 0, 0)
        for c in range(G):
            p = c % 2
            base = base_s + c * K
            if c < G - 1:
                if c >= 1:
                    for w in d_out[c - 1]:
                        w.wait()
                d_in[c + 1] = issue_in(base + K, c + 1, 1 - p)
            for w in d_in[c]:
                w.wait()
            compute(p)
            d_out[c] = issue_out(base, c, p)
        for w in d_out[G - 2]:
            w.wait()
        for w in d_out[G - 1]:
            w.wait()
        return carry

    lax.fori_loop(jnp.int32(0), jnp.int32(NSUP), super_body, jnp.int32(0))

    plsc.subcore_barrier()

    @pl.when(sid == 0)
    def _dump():
        pltpu.sync_copy(aggs, agg_hbm.at[cid])


def kernel(h, e, graph, edge_index, sparse,
           WU, bU, WV, bV, WA, bA, WB, bB, WC, bC,
           gamma_h, beta_h, gamma_e, beta_e):
    src = edge_index[0].astype(jnp.int32)
    dst = edge_index[1].astype(jnp.int32)

    W4 = jnp.concatenate([WU, WA, WB, WV], axis=1)
    b4 = jnp.concatenate([bU, bA, bB, bV]).reshape(1, H4)
    bC2 = bC.reshape(1, H)

    uh, ah, bh, vh = _node_mm(h, W4, b4)
    ce = _ce_mm(e, WC, bC2)

    zeros = jnp.zeros((V, H), _f32)
    enew, agg = _sc_edge_kernel(ah, vh, bh, ce, src, dst, zeros)

    h_out = _h_final(h, uh, agg, gamma_h.reshape(1, H), beta_h.reshape(1, H))
    e_out = _e_final(e, enew, gamma_e.reshape(1, H), beta_e.reshape(1, H))
    return (h_out, e_out)
